# baseline (speedup 1.0000x reference)
import functools

import jax
import jax.numpy as jnp
from jax import lax
from jax.experimental import pallas as pl
from jax.experimental.pallas import tpu as pltpu
from jax.experimental.pallas import tpu_sc as plsc

S = 128
B = 4096
D = 64
V = 1000000
NSLAB = 2
HS = V // NSLAB
N = S * B
NC, NS = 2, 16
NW = NC * NS
RPW = N // NW
CH = 256
NSUB = CH // 128
NCHUNK = RPW // CH
LN_EPS = 1e-5
EPS_SMALL = LN_EPS / 64.0
MAGIC = 0x5F3759DF

_MESH = dict(core_axis_name="c", subcore_axis_name="s",
             num_cores=NC, num_subcores=NS)


def _gather_body(x_hbm, tab_hbm, pos_hbm, gam_hbm, bet_hbm, out_hbm,
                 idx0, idx1, rows0, rows1, ob0, ob1, pos_v, gam_v, bet_v,
                 sem_i, sem_g, sem_o):
    wid = lax.axis_index("s") * NC + lax.axis_index("c")
    idx = (idx0, idx1)
    rows = (rows0, rows1)
    outb = (ob0, ob1)

    pltpu.sync_copy(pos_hbm, pos_v)
    pltpu.sync_copy(gam_hbm, gam_v)
    pltpu.sync_copy(bet_hbm, bet_v)
    gk = [gam_v[pl.ds(16 * k, 16)] for k in range(4)]
    bk = [bet_v[pl.ds(16 * k, 16)] for k in range(4)]

    def start_idx(gi, b):
        base = wid * RPW + gi * CH
        s_idx = base // B
        col = base % B
        for j in range(NSUB):
            pltpu.async_copy(x_hbm.at[s_idx, pl.ds(col + j * 128, 128)],
                             idx[b].at[j], sem_i)

    def wait_idx(b):
        for j in range(NSUB):
            pltpu.make_async_copy(x_hbm.at[0, pl.ds(0, 128)],
                                  idx[b].at[j], sem_i).wait()

    def remap_idx(b):
        for j in range(NSUB):
            for t in range(8):
                i = idx[b][j, pl.ds(16 * t, 16)]
                q = i // HS
                idx[b][j, pl.ds(16 * t, 16)] = i * NSLAB - q * (V - 1)

    def start_gather(b):
        for j in range(NSUB):
            pltpu.async_copy(tab_hbm.at[idx[b].at[j]],
                             rows[b].at[pl.ds(j * 128, 128)], sem_g)

    def wait_gather(b):
        for j in range(NSUB):
            pltpu.make_async_copy(tab_hbm.at[idx[b].at[j]],
                                  rows[b].at[pl.ds(j * 128, 128)],
                                  sem_g).wait()

    def start_wb(gi, b):
        base = wid * RPW + gi * CH
        pltpu.async_copy(outb[b], out_hbm.at[pl.ds(base * 128, CH * 128)],
                         sem_o)

    def wait_wb(b):
        pltpu.make_async_copy(out_hbm.at[pl.ds(0, CH * 128)], outb[b],
                              sem_o).wait()

    def compute(gi, b):
        s_idx = (wid * RPW + gi * CH) // B
        pk = [pos_v[s_idx, pl.ds(16 * k, 16)] * 0.125 for k in range(4)]
        rv = rows[b]
        ob = outb[b]

        @plsc.parallel_loop(0, CH, unroll=4)
        def _row(r):
            v = [rv[r, pl.ds(16 * k, 16)] + pk[k] for k in range(4)]
            sv = (v[0] + v[1]) + (v[2] + v[3])
            qv = (v[0] * v[0] + v[1] * v[1]) + (v[2] * v[2] + v[3] * v[3])
            mean = jnp.sum(sv) * (1.0 / 64.0)
            var = jnp.sum(qv) * (1.0 / 64.0) - mean * mean + EPS_SMALL
            iv = lax.bitcast_convert_type(var, jnp.int32)
            y = lax.bitcast_convert_type(MAGIC - (iv >> 1), jnp.float32)
            y = y * (1.5 - 0.5 * var * y * y)
            y = y * (1.5 - 0.5 * var * y * y)
            y = y * (1.5 - 0.5 * var * y * y)
            for k in range(4):
                ob[pl.ds(r * 128 + 16 * k, 16)] = (
                    (v[k] - mean) * y * gk[k] + bk[k])

    start_idx(0, 0)
    start_idx(1, 1)
    wait_idx(0)
    remap_idx(0)
    start_gather(0)

    @pl.loop(0, NCHUNK, step=2)
    def _chunks(g):
        for p in range(2):
            gi = g + p
            b = p
            wait_gather(b)

            @pl.when(gi + 2 < NCHUNK)
            def _():
                start_idx(gi + 2, b)

            @pl.when(gi >= 1)
            def _():
                wait_wb(1 - b)

            @pl.when(gi + 1 < NCHUNK)
            def _():
                wait_idx(1 - b)
                remap_idx(1 - b)
                start_gather(1 - b)

            compute(gi, b)
            start_wb(gi, b)

    wait_wb(1)


@functools.partial(jax.jit, static_argnames=())
def kernel(x, token_table, pos_table, ln_gamma, ln_beta):
    tab128 = jnp.concatenate([token_table[:HS], token_table[HS:]], axis=1)
    tab_lin = tab128.reshape(V, D)

    call = pl.kernel(
        _gather_body,
        out_type=jax.ShapeDtypeStruct((S * B * 128,), jnp.float32),
        mesh=plsc.VectorSubcoreMesh(**_MESH),
        scratch_types=[
            pltpu.VMEM((NSUB, 128), jnp.int32),
            pltpu.VMEM((NSUB, 128), jnp.int32),
            pltpu.VMEM((CH, D), jnp.float32),
            pltpu.VMEM((CH, D), jnp.float32),
            pltpu.VMEM((CH * 128,), jnp.float32),
            pltpu.VMEM((CH * 128,), jnp.float32),
            pltpu.VMEM((S, D), jnp.float32),
            pltpu.VMEM((D,), jnp.float32),
            pltpu.VMEM((D,), jnp.float32),
            pltpu.SemaphoreType.DMA,
            pltpu.SemaphoreType.DMA,
            pltpu.SemaphoreType.DMA,
        ],
        compiler_params=pltpu.CompilerParams(
            needs_layout_passes=False, use_tc_tiling_on_sc=False),
    )
    flat = call(x, tab_lin, pos_table, ln_gamma, ln_beta)
    return flat.reshape(S, B, 128)[:, :, :D]
